# baseline (speedup 1.0000x reference)
import functools

import jax
import jax.numpy as jnp
from jax import lax
from jax.experimental import pallas as pl
from jax.experimental.pallas import tpu as pltpu
from jax.experimental.pallas import tpu_sc as plsc

_K = 8192
_D = 256
_N = 16384
_BN = 1024
_BETA = 0.25


def _dist_argmin_body(f2_ref, flat_ref, w_ref, w2_ref, inds_ref, loss_ref):
    i = pl.program_id(0)
    mm = lax.dot_general(flat_ref[...], w_ref[...],
                         (((1,), (1,)), ((), ())))
    dist = (f2_ref[...] + w2_ref[...]) + mm
    minv = jnp.min(dist, axis=1, keepdims=True)
    iota = lax.broadcasted_iota(jnp.int32, dist.shape, 1)
    inds_ref[...] = jnp.min(jnp.where(dist == minv, iota, _K), axis=1)

    @pl.when(i == 0)
    def _init():
        loss_ref[...] = jnp.zeros_like(loss_ref)

    loss_ref[...] += jnp.sum(minv).reshape(1, 1)


def _tc_dist_argmin(f2, flat, weight, w2):
    return pl.pallas_call(
        _dist_argmin_body,
        grid=(_N // _BN,),
        in_specs=[
            pl.BlockSpec((_BN, 1), lambda i: (i, 0)),
            pl.BlockSpec((_BN, _D), lambda i: (i, 0)),
            pl.BlockSpec((_K, _D), lambda i: (0, 0)),
            pl.BlockSpec((1, _K), lambda i: (0, 0)),
        ],
        out_specs=[
            pl.BlockSpec((_BN,), lambda i: (i,)),
            pl.BlockSpec((1, 1), lambda i: (0, 0)),
        ],
        out_shape=[
            jax.ShapeDtypeStruct((_N,), jnp.int32),
            jax.ShapeDtypeStruct((1, 1), jnp.float32),
        ],
    )(f2, flat, weight, w2)


_CH = 128


@functools.lru_cache(maxsize=1)
def _make_sc_gather():
    info = plsc.get_sparse_core_info()
    nw = info.num_cores * info.num_subcores
    per_w = _N // nw
    n_ch = per_w // _CH
    mesh = plsc.VectorSubcoreMesh(core_axis_name="c", subcore_axis_name="s")

    @functools.partial(
        pl.kernel,
        mesh=mesh,
        out_type=jax.ShapeDtypeStruct((_N, _D), jnp.float32),
        scratch_types=[
            pltpu.VMEM((_CH,), jnp.int32),
            pltpu.VMEM((_CH, _D), jnp.float32),
            pltpu.SemaphoreType.DMA,
        ],
    )
    def gather(table_hbm, idx_hbm, out_hbm, idx_v, rows_v, sem):
        wid = lax.axis_index("s") * info.num_cores + lax.axis_index("c")
        for c in range(n_ch):
            base = wid * per_w + c * _CH
            pltpu.sync_copy(idx_hbm.at[pl.ds(base, _CH)], idx_v)
            pltpu.async_copy(table_hbm.at[idx_v], rows_v, sem).wait()
            pltpu.sync_copy(rows_v, out_hbm.at[pl.ds(base, _CH)])

    return gather


def kernel(latents, weight):
    lat = jnp.transpose(latents, (0, 2, 3, 1))
    shape = lat.shape
    flat = lat.reshape(-1, _D)
    f2 = jnp.sum(flat ** 2, axis=1, keepdims=True)
    w2 = jnp.sum(weight ** 2, axis=1)[None, :]
    inds, loss_sum = _tc_dist_argmin(f2, flat * -2.0, weight, w2)
    q_flat = _make_sc_gather()(weight, inds)
    vq_loss = loss_sum[0, 0] * ((1.0 + _BETA) / (_N * _D))
    quantized = q_flat.reshape(shape)
    return (jnp.transpose(quantized, (0, 3, 1, 2)), vq_loss)


# speedup vs baseline: 1.1651x; 1.1651x over previous
import functools

import jax
import jax.numpy as jnp
from jax import lax
from jax.experimental import pallas as pl
from jax.experimental.pallas import tpu as pltpu
from jax.experimental.pallas import tpu_sc as plsc

_K = 8192
_D = 256
_N = 16384
_BN = 1024
_BETA = 0.25


def _dist_argmin_body(f2_ref, flat_ref, w_ref, w2_ref, inds_ref, loss_ref):
    i = pl.program_id(0)
    mm = lax.dot_general(flat_ref[...], w_ref[...],
                         (((1,), (1,)), ((), ())))
    dist = (f2_ref[...] + w2_ref[...]) - 2.0 * mm
    minv = jnp.min(dist, axis=1, keepdims=True)
    iota = lax.broadcasted_iota(jnp.int32, dist.shape, 1)
    inds_ref[...] = jnp.min(jnp.where(dist == minv, iota, _K), axis=1)

    @pl.when(i == 0)
    def _init():
        loss_ref[...] = jnp.zeros_like(loss_ref)

    loss_ref[...] += jnp.sum(minv).reshape(1, 1)


def _tc_dist_argmin(f2, flat, weight, w2):
    return pl.pallas_call(
        _dist_argmin_body,
        grid=(_N // _BN,),
        in_specs=[
            pl.BlockSpec((_BN, 1), lambda i: (i, 0)),
            pl.BlockSpec((_BN, _D), lambda i: (i, 0)),
            pl.BlockSpec((_K, _D), lambda i: (0, 0)),
            pl.BlockSpec((1, _K), lambda i: (0, 0)),
        ],
        out_specs=[
            pl.BlockSpec((_BN,), lambda i: (i,)),
            pl.BlockSpec((1, 1), lambda i: (0, 0)),
        ],
        out_shape=[
            jax.ShapeDtypeStruct((_N,), jnp.int32),
            jax.ShapeDtypeStruct((1, 1), jnp.float32),
        ],
    )(f2, flat, weight, w2)


_CH = 128


@functools.lru_cache(maxsize=1)
def _make_sc_gather():
    info = plsc.get_sparse_core_info()
    nw = info.num_cores * info.num_subcores
    per_w = _N // nw
    n_ch = per_w // _CH
    mesh = plsc.VectorSubcoreMesh(core_axis_name="c", subcore_axis_name="s")

    @functools.partial(
        pl.kernel,
        mesh=mesh,
        out_type=jax.ShapeDtypeStruct((_N, _D), jnp.float32),
        scratch_types=[
            pltpu.VMEM((_CH,), jnp.int32),
            pltpu.VMEM((_CH, _D), jnp.float32),
            pltpu.SemaphoreType.DMA,
        ],
    )
    def gather(table_hbm, idx_hbm, out_hbm, idx_v, rows_v, sem):
        wid = lax.axis_index("s") * info.num_cores + lax.axis_index("c")
        for c in range(n_ch):
            base = wid * per_w + c * _CH
            pltpu.sync_copy(idx_hbm.at[pl.ds(base, _CH)], idx_v)
            pltpu.async_copy(table_hbm.at[idx_v], rows_v, sem).wait()
            pltpu.sync_copy(rows_v, out_hbm.at[pl.ds(base, _CH)])

    return gather


def kernel(latents, weight):
    lat = jnp.transpose(latents, (0, 2, 3, 1))
    shape = lat.shape
    flat = lat.reshape(-1, _D)
    f2 = jnp.sum(flat ** 2, axis=1, keepdims=True)
    w2 = jnp.sum(weight ** 2, axis=1)[None, :]
    inds, loss_sum = _tc_dist_argmin(f2, flat, weight, w2)
    q_flat = _make_sc_gather()(weight, inds)
    vq_loss = loss_sum[0, 0] * ((1.0 + _BETA) / (_N * _D))
    quantized = q_flat.reshape(shape)
    return (jnp.transpose(quantized, (0, 3, 1, 2)), vq_loss)
